# Initial kernel scaffold; baseline (speedup 1.0000x reference)
import jax
import jax.numpy as jnp
from jax.experimental import pallas as pl


def kernel(x, gate_w, expert_w, expert_b, agg_w, agg_b, orig_w, orig_b):
    raise NotImplementedError("write your pallas kernel here")


# speedup vs baseline: 4.7931x; 4.7931x over previous
import jax
import jax.numpy as jnp
from jax.experimental import pallas as pl

_E, _O, _D = 64, 192, 768
_TT = 256


_ET = 8


def _vred_body(aggw_ref, ew_ref, v_ref):
    a = aggw_ref[...]
    for e in range(_ET):
        v_ref[e, :] = jax.lax.dot_general(
            a, ew_ref[e],
            dimension_numbers=(((1,), (0,)), ((), ())),
            preferred_element_type=jnp.float32)[0]


def _tok_body(x_ref, gw_ref, v_ref, eb_ref, aggw_ref, ow_ref, ob_ref, ab_ref,
              out_ref):
    x = x_ref[...]
    logits = jax.lax.dot_general(
        x, gw_ref[...], (((1,), (1,)), ((), ())),
        preferred_element_type=jnp.float32)
    m = jnp.max(logits, axis=1, keepdims=True)
    iota = jax.lax.broadcasted_iota(jnp.int32, logits.shape, 1)
    idx = jnp.min(jnp.where(logits == m, iota, _E), axis=1, keepdims=True)
    oh = (iota == idx).astype(jnp.float32)
    vsel = jax.lax.dot_general(
        oh, v_ref[...], (((1,), (0,)), ((), ())),
        preferred_element_type=jnp.float32)
    cvec = jnp.sum(eb_ref[...] * aggw_ref[...], axis=1)
    csel = jnp.sum(oh * cvec[None, :], axis=1)
    w = vsel + ow_ref[...]
    dots = jnp.sum(x * w, axis=1)
    out_ref[...] = (dots + csel + ob_ref[0, 0] + ab_ref[0, 0])[:, None]


def kernel(x, gate_w, expert_w, expert_b, agg_w, agg_b, orig_w, orig_b):
    B, S, D = x.shape
    G = agg_w.shape[0]
    T = B * S
    hs = x.reshape(T, D)

    v = pl.pallas_call(
        _vred_body,
        grid=(_E // _ET,),
        in_specs=[pl.BlockSpec((1, _O), lambda e: (0, 0)),
                  pl.BlockSpec((_ET, _O, _D), lambda e: (e, 0, 0))],
        out_specs=pl.BlockSpec((_ET, _D), lambda e: (e, 0)),
        out_shape=jax.ShapeDtypeStruct((_E, _D), jnp.float32),
    )(agg_w, expert_w)

    ob = orig_b.reshape(1, 1)
    ab = agg_b.reshape(1, 1)
    out = pl.pallas_call(
        _tok_body,
        grid=(T // _TT,),
        in_specs=[pl.BlockSpec((_TT, _D), lambda i: (i, 0)),
                  pl.BlockSpec((_E, _D), lambda i: (0, 0)),
                  pl.BlockSpec((_E, _D), lambda i: (0, 0)),
                  pl.BlockSpec((_E, _O), lambda i: (0, 0)),
                  pl.BlockSpec((1, _O), lambda i: (0, 0)),
                  pl.BlockSpec((1, _D), lambda i: (0, 0)),
                  pl.BlockSpec((1, 1), lambda i: (0, 0)),
                  pl.BlockSpec((1, 1), lambda i: (0, 0))],
        out_specs=pl.BlockSpec((_TT, 1), lambda i: (i, 0)),
        out_shape=jax.ShapeDtypeStruct((T, 1), jnp.float32),
    )(hs, gate_w, v, expert_b, agg_w, orig_w, ob, ab)

    return out.reshape(B, S, G)


# speedup vs baseline: 4.8546x; 1.0128x over previous
import jax
import jax.numpy as jnp
from jax.experimental import pallas as pl
from jax.experimental.pallas import tpu as pltpu

_E, _O, _D = 64, 192, 768
_ET = 8
_TT = 256
_NE = _E // _ET


def _body(aggw_ref, ew_ref, x_ref, gw_ref, eb_ref, ow_ref, ob_ref, ab_ref,
          out_ref, v_scr):
    i = pl.program_id(0)

    @pl.when(i < _NE)
    def _vred():
        a = aggw_ref[...]
        a_rep = jnp.concatenate([a] * _ET, axis=1)
        rows = jax.lax.broadcasted_iota(jnp.int32, (_ET, _ET * _O), 0)
        cols = jax.lax.broadcasted_iota(jnp.int32, (_ET, _ET * _O), 1)
        amat = jnp.where(rows == cols // _O,
                         jnp.broadcast_to(a_rep, (_ET, _ET * _O)), 0.0)
        v_scr[pl.ds(i * _ET, _ET), :] = jax.lax.dot_general(
            amat, ew_ref[...], (((1,), (0,)), ((), ())),
            preferred_element_type=jnp.float32)

    @pl.when(i >= _NE)
    def _tokens():
        x = x_ref[...]
        logits = jax.lax.dot_general(
            x, gw_ref[...], (((1,), (1,)), ((), ())),
            preferred_element_type=jnp.float32)
        m = jnp.max(logits, axis=1, keepdims=True)
        iota = jax.lax.broadcasted_iota(jnp.int32, logits.shape, 1)
        idx = jnp.min(jnp.where(logits == m, iota, _E), axis=1, keepdims=True)
        oh = (iota == idx).astype(jnp.float32)
        vsel = jax.lax.dot_general(
            oh, v_scr[...], (((1,), (0,)), ((), ())),
            preferred_element_type=jnp.float32)
        cvec = jnp.sum(eb_ref[...] * aggw_ref[...], axis=1)
        csel = jnp.sum(oh * cvec[None, :], axis=1)
        w = vsel + ow_ref[...]
        dots = jnp.sum(x * w, axis=1)
        out_ref[...] = (dots + csel + ob_ref[0, 0] + ab_ref[0, 0])[:, None]


def kernel(x, gate_w, expert_w, expert_b, agg_w, agg_b, orig_w, orig_b):
    B, S, D = x.shape
    G = agg_w.shape[0]
    T = B * S
    hs = x.reshape(T, D)
    ob = orig_b.reshape(1, 1)
    ab = agg_b.reshape(1, 1)
    nt = T // _TT
    last_e = _NE - 1

    out = pl.pallas_call(
        _body,
        grid=(_NE + nt,),
        in_specs=[
            pl.BlockSpec((1, _O), lambda i: (0, 0)),
            pl.BlockSpec((_ET * _O, _D),
                         lambda i: (jnp.minimum(i, last_e), 0)),
            pl.BlockSpec((_TT, _D),
                         lambda i: (jnp.maximum(i - _NE, 0), 0)),
            pl.BlockSpec((_E, _D), lambda i: (0, 0)),
            pl.BlockSpec((_E, _O), lambda i: (0, 0)),
            pl.BlockSpec((1, _D), lambda i: (0, 0)),
            pl.BlockSpec((1, 1), lambda i: (0, 0)),
            pl.BlockSpec((1, 1), lambda i: (0, 0)),
        ],
        out_specs=pl.BlockSpec((_TT, 1), lambda i: (jnp.maximum(i - _NE, 0), 0)),
        out_shape=jax.ShapeDtypeStruct((T, 1), jnp.float32),
        scratch_shapes=[pltpu.VMEM((_E, _D), jnp.float32)],
    )(agg_w, expert_w.reshape(_E * _O, D), hs, gate_w, expert_b, orig_w,
      ob, ab)

    return out.reshape(B, S, G)


# speedup vs baseline: 4.8773x; 1.0047x over previous
import jax
import jax.numpy as jnp
from jax.experimental import pallas as pl
from jax.experimental.pallas import tpu as pltpu

_E, _O, _D = 64, 192, 768
_ET = 8
_TT = 256
_NE = _E // _ET


def _body(aggw_ref, ew_ref, x_ref, gw_ref, eb_ref, ow_ref, ob_ref, ab_ref,
          out_ref, v_scr, amat_scr):
    i = pl.program_id(0)

    @pl.when(i == 0)
    def _amat():
        a = aggw_ref[...]
        a_rep = jnp.concatenate([a] * _ET, axis=1)
        rows = jax.lax.broadcasted_iota(jnp.int32, (_ET, _ET * _O), 0)
        cols = jax.lax.broadcasted_iota(jnp.int32, (_ET, _ET * _O), 1)
        amat_scr[...] = jnp.where(rows == cols // _O,
                                  jnp.broadcast_to(a_rep, (_ET, _ET * _O)),
                                  0.0)

    @pl.when(i < _NE)
    def _vred():
        v_scr[pl.ds(i * _ET, _ET), :] = jax.lax.dot_general(
            amat_scr[...], ew_ref[...], (((1,), (0,)), ((), ())),
            preferred_element_type=jnp.float32)

    @pl.when(i >= _NE)
    def _tokens():
        x = x_ref[...]
        logits = jax.lax.dot_general(
            x, gw_ref[...], (((1,), (1,)), ((), ())),
            preferred_element_type=jnp.float32)
        m = jnp.max(logits, axis=1, keepdims=True)
        iota = jax.lax.broadcasted_iota(jnp.int32, logits.shape, 1)
        idx = jnp.min(jnp.where(logits == m, iota, _E), axis=1, keepdims=True)
        oh = (iota == idx).astype(jnp.float32)
        vsel = jax.lax.dot_general(
            oh, v_scr[...], (((1,), (0,)), ((), ())),
            preferred_element_type=jnp.float32)
        cvec = jnp.sum(eb_ref[...] * aggw_ref[...], axis=1)
        csel = jnp.sum(oh * cvec[None, :], axis=1)
        w = vsel + ow_ref[...]
        dots = jnp.sum(x * w, axis=1)
        out_ref[...] = (dots + csel + ob_ref[0, 0] + ab_ref[0, 0])[:, None]


def kernel(x, gate_w, expert_w, expert_b, agg_w, agg_b, orig_w, orig_b):
    B, S, D = x.shape
    G = agg_w.shape[0]
    T = B * S
    hs = x.reshape(T, D)
    ob = orig_b.reshape(1, 1)
    ab = agg_b.reshape(1, 1)
    nt = T // _TT
    last_e = _NE - 1

    out = pl.pallas_call(
        _body,
        grid=(_NE + nt,),
        in_specs=[
            pl.BlockSpec((1, _O), lambda i: (0, 0)),
            pl.BlockSpec((_ET * _O, _D),
                         lambda i: (jnp.minimum(i, last_e), 0)),
            pl.BlockSpec((_TT, _D),
                         lambda i: (jnp.maximum(i - _NE, 0), 0)),
            pl.BlockSpec((_E, _D), lambda i: (0, 0)),
            pl.BlockSpec((_E, _O), lambda i: (0, 0)),
            pl.BlockSpec((1, _D), lambda i: (0, 0)),
            pl.BlockSpec((1, 1), lambda i: (0, 0)),
            pl.BlockSpec((1, 1), lambda i: (0, 0)),
        ],
        out_specs=pl.BlockSpec((_TT, 1), lambda i: (jnp.maximum(i - _NE, 0), 0)),
        out_shape=jax.ShapeDtypeStruct((T, 1), jnp.float32),
        scratch_shapes=[pltpu.VMEM((_E, _D), jnp.float32),
                        pltpu.VMEM((_ET, _ET * _O), jnp.float32)],
    )(agg_w, expert_w.reshape(_E * _O, D), hs, gate_w, expert_b, orig_w,
      ob, ab)

    return out.reshape(B, S, G)
